# Initial kernel scaffold; baseline (speedup 1.0000x reference)
import jax
import jax.numpy as jnp
from jax.experimental import pallas as pl


def kernel(nodes, edges, W_emb, b_emb, W_att, b_att, Wz, Uz, bz, Wr, Ur, br, Wh, Uh, bh, W_gate, b_gate, W_er, b_er, W_out, b_out):
    raise NotImplementedError("write your pallas kernel here")


# speedup vs baseline: 3.5692x; 3.5692x over previous
import jax
import jax.numpy as jnp
from jax.experimental import pallas as pl
from jax.experimental.pallas import tpu as pltpu

_B, _N, _H, _E, _MSG, _PASSES = 16, 64, 128, 16, 128, 3
_BIG_NEG = -1000000.0
_F32 = jnp.float32


def _mpnn_body(nodes_ref, edges_ref, W_emb_ref, b_emb_ref, W_att_ref,
               b_att_ref, Wz_ref, Uz_ref, bz_ref, Wr_ref, Ur_ref, br_ref,
               Wh_ref, Uh_ref, bh_ref, W_gate_ref, b_gate_ref, W_er_ref,
               b_er_ref, W_out_ref, b_out_ref, out_ref):
    n = _N
    nodes = nodes_ref[0]
    edges = edges_ref[0]

    adjacency = jnp.sum(edges, axis=-1)
    nghb_mask = (adjacency != 0).astype(_F32)
    node_mask = jnp.sum(adjacency, axis=-1, keepdims=True) != 0

    e2 = edges.reshape(n * n, _E)
    ep_emb = jnp.dot(e2, W_emb_ref[_H:, :],
                     preferred_element_type=_F32).reshape(n, n, _MSG)
    ep_att = jnp.dot(e2, W_att_ref[_H:, :],
                     preferred_element_type=_F32).reshape(n, n, _MSG)

    b_emb = b_emb_ref[0]
    b_att = b_att_ref[0]
    mask3 = nghb_mask[:, :, None]

    hidden = nodes
    for _ in range(_PASSES):
        hp = jnp.dot(hidden, W_emb_ref[:_H, :],
                     preferred_element_type=_F32) + b_emb
        ha = jnp.dot(hidden, W_att_ref[:_H, :],
                     preferred_element_type=_F32) + b_att

        energy = ha[None, :, :] + ep_att
        energy = jnp.where(mask3 > 0, energy, _BIG_NEG)
        mx = jnp.max(energy, axis=1, keepdims=True)
        p = jnp.exp(energy - mx)
        s = jnp.sum(p, axis=1, keepdims=True)
        emb = jnp.maximum(hp[None, :, :] + ep_emb, 0.0)
        msgs = jnp.sum((p / s) * emb * mask3, axis=1)

        z = jax.nn.sigmoid(
            jnp.dot(msgs, Wz_ref[...], preferred_element_type=_F32)
            + jnp.dot(hidden, Uz_ref[...], preferred_element_type=_F32)
            + bz_ref[0])
        r = jax.nn.sigmoid(
            jnp.dot(msgs, Wr_ref[...], preferred_element_type=_F32)
            + jnp.dot(hidden, Ur_ref[...], preferred_element_type=_F32)
            + br_ref[0])
        htil = jnp.tanh(
            jnp.dot(msgs, Wh_ref[...], preferred_element_type=_F32)
            + jnp.dot(r * hidden, Uh_ref[...], preferred_element_type=_F32)
            + bh_ref[0])
        hidden_new = (1.0 - z) * hidden + z * htil
        hidden = jnp.where(node_mask, hidden_new, hidden)

    gate = jax.nn.sigmoid(
        jnp.dot(hidden, W_gate_ref[:_H, :], preferred_element_type=_F32)
        + jnp.dot(nodes, W_gate_ref[_H:, :], preferred_element_type=_F32)
        + b_gate_ref[0])
    embr = jnp.dot(hidden, W_er_ref[...],
                   preferred_element_type=_F32) + b_er_ref[0]
    nm = node_mask.astype(_F32)
    graph_emb = jnp.sum(gate * embr * nm, axis=0, keepdims=True)
    out = jnp.dot(jnp.maximum(graph_emb, 0.0), W_out_ref[...],
                  preferred_element_type=_F32) + b_out_ref[0]
    out_ref[0] = out


def kernel(nodes, edges, W_emb, b_emb, W_att, b_att, Wz, Uz, bz, Wr, Ur, br,
           Wh, Uh, bh, W_gate, b_gate, W_er, b_er, W_out, b_out):
    def full(shape):
        return pl.BlockSpec(shape, lambda b: (0,) * len(shape))

    biases = [b.reshape(1, -1) for b in (b_emb, b_att, bz, br, bh,
                                         b_gate, b_er, b_out)]
    (b_emb2, b_att2, bz2, br2, bh2, b_gate2, b_er2, b_out2) = biases

    grid_spec = pl.GridSpec(
        grid=(_B,),
        in_specs=[
            pl.BlockSpec((1, _N, _H), lambda b: (b, 0, 0)),
            pl.BlockSpec((1, _N, _N, _E), lambda b: (b, 0, 0, 0)),
            full(W_emb.shape), full(b_emb2.shape),
            full(W_att.shape), full(b_att2.shape),
            full(Wz.shape), full(Uz.shape), full(bz2.shape),
            full(Wr.shape), full(Ur.shape), full(br2.shape),
            full(Wh.shape), full(Uh.shape), full(bh2.shape),
            full(W_gate.shape), full(b_gate2.shape),
            full(W_er.shape), full(b_er2.shape),
            full(W_out.shape), full(b_out2.shape),
        ],
        out_specs=pl.BlockSpec((1, 1, _H), lambda b: (b, 0, 0)),
    )

    out = pl.pallas_call(
        _mpnn_body,
        grid_spec=grid_spec,
        out_shape=jax.ShapeDtypeStruct((_B, 1, _H), _F32),
        compiler_params=pltpu.CompilerParams(
            dimension_semantics=("parallel",)),
    )(nodes, edges, W_emb, b_emb2, W_att, b_att2, Wz, Uz, bz2,
      Wr, Ur, br2, Wh, Uh, bh2, W_gate, b_gate2, W_er, b_er2,
      W_out, b_out2)
    return out.reshape(_B, _H)


# speedup vs baseline: 3.9929x; 1.1187x over previous
import jax
import jax.numpy as jnp
from jax.experimental import pallas as pl
from jax.experimental.pallas import tpu as pltpu

_B, _N, _H, _E, _MSG, _PASSES = 16, 64, 128, 16, 128, 3
_F32 = jnp.float32


def _mpnn_body(nodes_ref, e2_ref, W_edge_ref, W_node_ref, b_na_ref,
               W_zr_ref, U_zr_ref, b_zr_ref, Wh_ref, Uh_ref, bh_ref,
               W_gate_ref, b_gate_ref, W_er_ref, b_er_ref,
               W_out_ref, b_out_ref, out_ref):
    n = _N
    nodes = nodes_ref[0]
    e2 = e2_ref[0]

    ep = jnp.dot(e2, W_edge_ref[...], preferred_element_type=_F32)
    ep_emb = ep[:, :_MSG].reshape(n, n, _MSG)
    ep_att = ep[:, _MSG:2 * _MSG].reshape(n, n, _MSG)
    madj = ep[:, 2 * _MSG:].reshape(n, n, _MSG)
    maskf = jnp.where(madj != 0.0, 1.0, 0.0)
    node_mask = jnp.sum(madj, axis=1) != 0.0

    b_na = b_na_ref[0]
    b_zr = b_zr_ref[0]
    bh = bh_ref[0]
    hidden = nodes
    for _ in range(_PASSES):
        hn = jnp.dot(hidden, W_node_ref[...],
                     preferred_element_type=_F32) + b_na
        hp = hn[:, :_MSG]
        ha = hn[:, _MSG:]
        pm = jnp.exp(ha[None, :, :] + ep_att) * maskf
        emb = jnp.maximum(hp[None, :, :] + ep_emb, 0.0)
        s = jnp.sum(pm, axis=1)
        msum = jnp.sum(pm * emb, axis=1)
        msgs = msum / jnp.maximum(s, 1e-30)

        zr = jax.nn.sigmoid(
            jnp.dot(msgs, W_zr_ref[...], preferred_element_type=_F32)
            + jnp.dot(hidden, U_zr_ref[...], preferred_element_type=_F32)
            + b_zr)
        z = zr[:, :_H]
        r = zr[:, _H:]
        htil = jnp.tanh(
            jnp.dot(msgs, Wh_ref[...], preferred_element_type=_F32)
            + jnp.dot(r * hidden, Uh_ref[...], preferred_element_type=_F32)
            + bh)
        hidden_new = (1.0 - z) * hidden + z * htil
        hidden = jnp.where(node_mask, hidden_new, hidden)

    gate = jax.nn.sigmoid(
        jnp.dot(hidden, W_gate_ref[:_H, :], preferred_element_type=_F32)
        + jnp.dot(nodes, W_gate_ref[_H:, :], preferred_element_type=_F32)
        + b_gate_ref[0])
    embr = jnp.dot(hidden, W_er_ref[...],
                   preferred_element_type=_F32) + b_er_ref[0]
    gp = jnp.where(node_mask, gate * embr, 0.0)
    graph_emb = jnp.sum(gp, axis=0, keepdims=True)
    out = jnp.dot(jnp.maximum(graph_emb, 0.0), W_out_ref[...],
                  preferred_element_type=_F32) + b_out_ref[0]
    out_ref[0] = out


def kernel(nodes, edges, W_emb, b_emb, W_att, b_att, Wz, Uz, bz, Wr, Ur, br,
           Wh, Uh, bh, W_gate, b_gate, W_er, b_er, W_out, b_out):
    e2 = edges.reshape(_B, _N * _N, _E)
    W_edge = jnp.concatenate(
        [W_emb[_H:], W_att[_H:], jnp.ones((_E, _MSG), _F32)], axis=1)
    W_node = jnp.concatenate([W_emb[:_H], W_att[:_H]], axis=1)
    b_na = jnp.concatenate([b_emb, b_att]).reshape(1, 2 * _MSG)
    W_zr = jnp.concatenate([Wz, Wr], axis=1)
    U_zr = jnp.concatenate([Uz, Ur], axis=1)
    b_zr = jnp.concatenate([bz, br]).reshape(1, 2 * _H)
    bh2 = bh.reshape(1, _H)
    b_gate2 = b_gate.reshape(1, _H)
    b_er2 = b_er.reshape(1, _H)
    b_out2 = b_out.reshape(1, _H)

    def full(shape):
        return pl.BlockSpec(shape, lambda b: (0,) * len(shape))

    grid_spec = pl.GridSpec(
        grid=(_B,),
        in_specs=[
            pl.BlockSpec((1, _N, _H), lambda b: (b, 0, 0)),
            pl.BlockSpec((1, _N * _N, _E), lambda b: (b, 0, 0)),
            full(W_edge.shape), full(W_node.shape), full(b_na.shape),
            full(W_zr.shape), full(U_zr.shape), full(b_zr.shape),
            full(Wh.shape), full(Uh.shape), full(bh2.shape),
            full(W_gate.shape), full(b_gate2.shape),
            full(W_er.shape), full(b_er2.shape),
            full(W_out.shape), full(b_out2.shape),
        ],
        out_specs=pl.BlockSpec((1, 1, _H), lambda b: (b, 0, 0)),
    )

    out = pl.pallas_call(
        _mpnn_body,
        grid_spec=grid_spec,
        out_shape=jax.ShapeDtypeStruct((_B, 1, _H), _F32),
        compiler_params=pltpu.CompilerParams(
            dimension_semantics=("parallel",)),
    )(nodes, e2, W_edge, W_node, b_na, W_zr, U_zr, b_zr,
      Wh, Uh, bh2, W_gate, b_gate2, W_er, b_er2, W_out, b_out2)
    return out.reshape(_B, _H)


# speedup vs baseline: 4.2955x; 1.0758x over previous
import jax
import jax.numpy as jnp
from jax.experimental import pallas as pl
from jax.experimental.pallas import tpu as pltpu

_B, _N, _H, _E, _MSG, _PASSES = 16, 64, 128, 16, 128, 3
_F32 = jnp.float32


def _mpnn_body(nodes_ref, e2_ref, W_edge_ref, W_node_ref, b_na_ref,
               W_zr_ref, U_zr_ref, b_zr_ref, Wh_ref, Uh_ref, bh_ref,
               W_gate_ref, b_gate_ref, W_er_ref, b_er_ref,
               W_out_ref, b_out_ref, out_ref):
    n = _N
    nodes = nodes_ref[0]
    e2 = e2_ref[0]

    ep = jnp.dot(e2, W_edge_ref[...], preferred_element_type=_F32)
    ep_emb = ep[:, :_MSG].reshape(n, n, _MSG)
    madj = ep[:, 2 * _MSG:].reshape(n, n, _MSG)
    ep_att = jnp.where(madj != 0.0,
                       ep[:, _MSG:2 * _MSG].reshape(n, n, _MSG), -1e30)
    node_mask = jnp.sum(madj, axis=1) != 0.0

    b_na = b_na_ref[0]
    b_zr = b_zr_ref[0]
    bh = bh_ref[0]
    hidden = nodes
    for _ in range(_PASSES):
        hn = jnp.dot(hidden, W_node_ref[...],
                     preferred_element_type=_F32) + b_na
        hp = hn[:, :_MSG]
        ha = hn[:, _MSG:]
        pm = jnp.exp(ha[None, :, :] + ep_att)
        emb = jnp.maximum(hp[None, :, :] + ep_emb, 0.0)
        s = jnp.sum(pm, axis=1)
        msum = jnp.sum(pm * emb, axis=1)
        msgs = msum / jnp.maximum(s, 1e-30)

        zr = jax.nn.sigmoid(
            jnp.dot(msgs, W_zr_ref[...], preferred_element_type=_F32)
            + jnp.dot(hidden, U_zr_ref[...], preferred_element_type=_F32)
            + b_zr)
        z = zr[:, :_H]
        r = zr[:, _H:]
        htil = jnp.tanh(
            jnp.dot(msgs, Wh_ref[...], preferred_element_type=_F32)
            + jnp.dot(r * hidden, Uh_ref[...], preferred_element_type=_F32)
            + bh)
        hidden_new = (1.0 - z) * hidden + z * htil
        hidden = jnp.where(node_mask, hidden_new, hidden)

    gate = jax.nn.sigmoid(
        jnp.dot(hidden, W_gate_ref[:_H, :], preferred_element_type=_F32)
        + jnp.dot(nodes, W_gate_ref[_H:, :], preferred_element_type=_F32)
        + b_gate_ref[0])
    embr = jnp.dot(hidden, W_er_ref[...],
                   preferred_element_type=_F32) + b_er_ref[0]
    gp = jnp.where(node_mask, gate * embr, 0.0)
    graph_emb = jnp.sum(gp, axis=0, keepdims=True)
    out = jnp.dot(jnp.maximum(graph_emb, 0.0), W_out_ref[...],
                  preferred_element_type=_F32) + b_out_ref[0]
    out_ref[0] = out


def kernel(nodes, edges, W_emb, b_emb, W_att, b_att, Wz, Uz, bz, Wr, Ur, br,
           Wh, Uh, bh, W_gate, b_gate, W_er, b_er, W_out, b_out):
    e2 = edges.reshape(_B, _N * _N, _E)
    W_edge = jnp.concatenate(
        [W_emb[_H:], W_att[_H:], jnp.ones((_E, _MSG), _F32)], axis=1)
    W_node = jnp.concatenate([W_emb[:_H], W_att[:_H]], axis=1)
    b_na = jnp.concatenate([b_emb, b_att]).reshape(1, 2 * _MSG)
    W_zr = jnp.concatenate([Wz, Wr], axis=1)
    U_zr = jnp.concatenate([Uz, Ur], axis=1)
    b_zr = jnp.concatenate([bz, br]).reshape(1, 2 * _H)
    bh2 = bh.reshape(1, _H)
    b_gate2 = b_gate.reshape(1, _H)
    b_er2 = b_er.reshape(1, _H)
    b_out2 = b_out.reshape(1, _H)

    def full(shape):
        return pl.BlockSpec(shape, lambda b: (0,) * len(shape))

    grid_spec = pl.GridSpec(
        grid=(_B,),
        in_specs=[
            pl.BlockSpec((1, _N, _H), lambda b: (b, 0, 0)),
            pl.BlockSpec((1, _N * _N, _E), lambda b: (b, 0, 0)),
            full(W_edge.shape), full(W_node.shape), full(b_na.shape),
            full(W_zr.shape), full(U_zr.shape), full(b_zr.shape),
            full(Wh.shape), full(Uh.shape), full(bh2.shape),
            full(W_gate.shape), full(b_gate2.shape),
            full(W_er.shape), full(b_er2.shape),
            full(W_out.shape), full(b_out2.shape),
        ],
        out_specs=pl.BlockSpec((1, 1, _H), lambda b: (b, 0, 0)),
    )

    out = pl.pallas_call(
        _mpnn_body,
        grid_spec=grid_spec,
        out_shape=jax.ShapeDtypeStruct((_B, 1, _H), _F32),
        compiler_params=pltpu.CompilerParams(
            dimension_semantics=("arbitrary",)),
    )(nodes, e2, W_edge, W_node, b_na, W_zr, U_zr, b_zr,
      Wh, Uh, bh2, W_gate, b_gate2, W_er, b_er2, W_out, b_out2)
    return out.reshape(_B, _H)


# speedup vs baseline: 4.7981x; 1.1170x over previous
import jax
import jax.numpy as jnp
from jax.experimental import pallas as pl
from jax.experimental.pallas import tpu as pltpu

_B, _N, _H, _E, _MSG, _PASSES = 16, 64, 128, 16, 128, 3
_G = 2
_F32 = jnp.float32


def _mpnn_body(nodes_ref, e2_ref, W_edge_ref, W_node_ref, b_na_ref,
               W_zr_ref, U_zr_ref, b_zr_ref, Wh_ref, Uh_ref, bh_ref,
               W_gate_ref, b_gate_ref, W_er_ref, b_er_ref,
               W_out_ref, b_out_ref, out_ref):
    g, n = _G, _N
    nodes = nodes_ref[...].reshape(g * n, _H)
    e2 = e2_ref[...].reshape(g * n * n, _E)

    ep = jnp.dot(e2, W_edge_ref[...], preferred_element_type=_F32)
    ep_emb = ep[:, :_MSG].reshape(g, n, n, _MSG)
    madj = ep[:, 2 * _MSG:].reshape(g, n, n, _MSG)
    ep_att = jnp.where(madj != 0.0,
                       ep[:, _MSG:2 * _MSG].reshape(g, n, n, _MSG), -1e30)
    node_mask = (jnp.sum(madj, axis=2) != 0.0).reshape(g * n, _MSG)

    b_na = b_na_ref[0]
    b_zr = b_zr_ref[0]
    bh = bh_ref[0]
    hidden = nodes
    for _ in range(_PASSES):
        hn = jnp.dot(hidden, W_node_ref[...],
                     preferred_element_type=_F32) + b_na
        hp = hn[:, :_MSG].reshape(g, 1, n, _MSG)
        ha = hn[:, _MSG:].reshape(g, 1, n, _MSG)
        pm = jnp.exp(ha + ep_att)
        emb = jnp.maximum(hp + ep_emb, 0.0)
        s = jnp.sum(pm, axis=2)
        msum = jnp.sum(pm * emb, axis=2)
        msgs = (msum / jnp.maximum(s, 1e-30)).reshape(g * n, _MSG)

        zr = jax.nn.sigmoid(
            jnp.dot(msgs, W_zr_ref[...], preferred_element_type=_F32)
            + jnp.dot(hidden, U_zr_ref[...], preferred_element_type=_F32)
            + b_zr)
        z = zr[:, :_H]
        r = zr[:, _H:]
        htil = jnp.tanh(
            jnp.dot(msgs, Wh_ref[...], preferred_element_type=_F32)
            + jnp.dot(r * hidden, Uh_ref[...], preferred_element_type=_F32)
            + bh)
        hidden_new = (1.0 - z) * hidden + z * htil
        hidden = jnp.where(node_mask, hidden_new, hidden)

    gate = jax.nn.sigmoid(
        jnp.dot(hidden, W_gate_ref[:_H, :], preferred_element_type=_F32)
        + jnp.dot(nodes, W_gate_ref[_H:, :], preferred_element_type=_F32)
        + b_gate_ref[0])
    embr = jnp.dot(hidden, W_er_ref[...],
                   preferred_element_type=_F32) + b_er_ref[0]
    gp = jnp.where(node_mask, gate * embr, 0.0).reshape(g, n, _H)
    graph_emb = jnp.sum(gp, axis=1)
    out = jnp.dot(jnp.maximum(graph_emb, 0.0), W_out_ref[...],
                  preferred_element_type=_F32) + b_out_ref[0]
    out_ref[...] = out.reshape(g, 1, _H)


def kernel(nodes, edges, W_emb, b_emb, W_att, b_att, Wz, Uz, bz, Wr, Ur, br,
           Wh, Uh, bh, W_gate, b_gate, W_er, b_er, W_out, b_out):
    e2 = edges.reshape(_B, _N * _N, _E)
    W_edge = jnp.concatenate(
        [W_emb[_H:], W_att[_H:], jnp.ones((_E, _MSG), _F32)], axis=1)
    W_node = jnp.concatenate([W_emb[:_H], W_att[:_H]], axis=1)
    b_na = jnp.concatenate([b_emb, b_att]).reshape(1, 2 * _MSG)
    W_zr = jnp.concatenate([Wz, Wr], axis=1)
    U_zr = jnp.concatenate([Uz, Ur], axis=1)
    b_zr = jnp.concatenate([bz, br]).reshape(1, 2 * _H)
    bh2 = bh.reshape(1, _H)
    b_gate2 = b_gate.reshape(1, _H)
    b_er2 = b_er.reshape(1, _H)
    b_out2 = b_out.reshape(1, _H)

    def full(shape):
        return pl.BlockSpec(shape, lambda b: (0,) * len(shape))

    grid_spec = pl.GridSpec(
        grid=(_B // _G,),
        in_specs=[
            pl.BlockSpec((_G, _N, _H), lambda b: (b, 0, 0)),
            pl.BlockSpec((_G, _N * _N, _E), lambda b: (b, 0, 0)),
            full(W_edge.shape), full(W_node.shape), full(b_na.shape),
            full(W_zr.shape), full(U_zr.shape), full(b_zr.shape),
            full(Wh.shape), full(Uh.shape), full(bh2.shape),
            full(W_gate.shape), full(b_gate2.shape),
            full(W_er.shape), full(b_er2.shape),
            full(W_out.shape), full(b_out2.shape),
        ],
        out_specs=pl.BlockSpec((_G, 1, _H), lambda b: (b, 0, 0)),
    )

    out = pl.pallas_call(
        _mpnn_body,
        grid_spec=grid_spec,
        out_shape=jax.ShapeDtypeStruct((_B, 1, _H), _F32),
        compiler_params=pltpu.CompilerParams(
            dimension_semantics=("arbitrary",)),
    )(nodes, e2, W_edge, W_node, b_na, W_zr, U_zr, b_zr,
      Wh, Uh, bh2, W_gate, b_gate2, W_er, b_er2, W_out, b_out2)
    return out.reshape(_B, _H)


# speedup vs baseline: 4.9692x; 1.0357x over previous
import jax
import jax.numpy as jnp
from jax.experimental import pallas as pl
from jax.experimental.pallas import tpu as pltpu

_B, _N, _H, _E, _MSG, _PASSES = 16, 64, 128, 16, 128, 3
_G = 4
_F32 = jnp.float32


def _mpnn_body(nodes_ref, e2_ref, W_edge_ref, W_node_ref, b_na_ref,
               W_zr_ref, U_zr_ref, b_zr_ref, Wh_ref, Uh_ref, bh_ref,
               W_gate_ref, b_gate_ref, W_er_ref, b_er_ref,
               W_out_ref, b_out_ref, out_ref):
    g, n = _G, _N
    nodes = nodes_ref[...].reshape(g * n, _H)
    e2 = e2_ref[...].reshape(g * n * n, _E)

    ep = jnp.dot(e2, W_edge_ref[...], preferred_element_type=_F32)
    ep_emb = ep[:, :_MSG].reshape(g, n, n, _MSG)
    madj = ep[:, 2 * _MSG:].reshape(g, n, n, _MSG)
    ep_att = jnp.where(madj != 0.0,
                       ep[:, _MSG:2 * _MSG].reshape(g, n, n, _MSG), -1e30)
    node_mask = (jnp.sum(madj, axis=2) != 0.0).reshape(g * n, _MSG)

    b_na = b_na_ref[0]
    b_zr = b_zr_ref[0]
    bh = bh_ref[0]
    hidden = nodes
    for _ in range(_PASSES):
        hn = jnp.dot(hidden, W_node_ref[...],
                     preferred_element_type=_F32) + b_na
        hp = hn[:, :_MSG].reshape(g, 1, n, _MSG)
        ha = hn[:, _MSG:].reshape(g, 1, n, _MSG)
        pm = jnp.exp(ha + ep_att)
        emb = jnp.maximum(hp + ep_emb, 0.0)
        s = jnp.sum(pm, axis=2)
        msum = jnp.sum(pm * emb, axis=2)
        msgs = (msum / jnp.maximum(s, 1e-30)).reshape(g * n, _MSG)

        zr = jax.nn.sigmoid(
            jnp.dot(msgs, W_zr_ref[...], preferred_element_type=_F32)
            + jnp.dot(hidden, U_zr_ref[...], preferred_element_type=_F32)
            + b_zr)
        z = zr[:, :_H]
        r = zr[:, _H:]
        htil = jnp.tanh(
            jnp.dot(msgs, Wh_ref[...], preferred_element_type=_F32)
            + jnp.dot(r * hidden, Uh_ref[...], preferred_element_type=_F32)
            + bh)
        hidden_new = (1.0 - z) * hidden + z * htil
        hidden = jnp.where(node_mask, hidden_new, hidden)

    gate = jax.nn.sigmoid(
        jnp.dot(hidden, W_gate_ref[:_H, :], preferred_element_type=_F32)
        + jnp.dot(nodes, W_gate_ref[_H:, :], preferred_element_type=_F32)
        + b_gate_ref[0])
    embr = jnp.dot(hidden, W_er_ref[...],
                   preferred_element_type=_F32) + b_er_ref[0]
    gp = jnp.where(node_mask, gate * embr, 0.0).reshape(g, n, _H)
    graph_emb = jnp.sum(gp, axis=1)
    out = jnp.dot(jnp.maximum(graph_emb, 0.0), W_out_ref[...],
                  preferred_element_type=_F32) + b_out_ref[0]
    out_ref[...] = out.reshape(g, 1, _H)


def kernel(nodes, edges, W_emb, b_emb, W_att, b_att, Wz, Uz, bz, Wr, Ur, br,
           Wh, Uh, bh, W_gate, b_gate, W_er, b_er, W_out, b_out):
    e2 = edges.reshape(_B, _N * _N, _E)
    W_edge = jnp.concatenate(
        [W_emb[_H:], W_att[_H:], jnp.ones((_E, _MSG), _F32)], axis=1)
    W_node = jnp.concatenate([W_emb[:_H], W_att[:_H]], axis=1)
    b_na = jnp.concatenate([b_emb, b_att]).reshape(1, 2 * _MSG)
    W_zr = jnp.concatenate([Wz, Wr], axis=1)
    U_zr = jnp.concatenate([Uz, Ur], axis=1)
    b_zr = jnp.concatenate([bz, br]).reshape(1, 2 * _H)
    bh2 = bh.reshape(1, _H)
    b_gate2 = b_gate.reshape(1, _H)
    b_er2 = b_er.reshape(1, _H)
    b_out2 = b_out.reshape(1, _H)

    def full(shape):
        return pl.BlockSpec(shape, lambda b: (0,) * len(shape))

    grid_spec = pl.GridSpec(
        grid=(_B // _G,),
        in_specs=[
            pl.BlockSpec((_G, _N, _H), lambda b: (b, 0, 0)),
            pl.BlockSpec((_G, _N * _N, _E), lambda b: (b, 0, 0)),
            full(W_edge.shape), full(W_node.shape), full(b_na.shape),
            full(W_zr.shape), full(U_zr.shape), full(b_zr.shape),
            full(Wh.shape), full(Uh.shape), full(bh2.shape),
            full(W_gate.shape), full(b_gate2.shape),
            full(W_er.shape), full(b_er2.shape),
            full(W_out.shape), full(b_out2.shape),
        ],
        out_specs=pl.BlockSpec((_G, 1, _H), lambda b: (b, 0, 0)),
    )

    out = pl.pallas_call(
        _mpnn_body,
        grid_spec=grid_spec,
        out_shape=jax.ShapeDtypeStruct((_B, 1, _H), _F32),
        compiler_params=pltpu.CompilerParams(
            dimension_semantics=("arbitrary",)),
    )(nodes, e2, W_edge, W_node, b_na, W_zr, U_zr, b_zr,
      Wh, Uh, bh2, W_gate, b_gate2, W_er, b_er2, W_out, b_out2)
    return out.reshape(_B, _H)


# speedup vs baseline: 5.5279x; 1.1124x over previous
import jax
import jax.numpy as jnp
from jax.experimental import pallas as pl
from jax.experimental.pallas import tpu as pltpu

_B, _N, _H, _E, _MSG, _PASSES = 16, 64, 128, 16, 128, 3
_G = 4
_F32 = jnp.float32


def _mpnn_body(nodes_ref, e2_ref, W_edge_ref, W_node_ref, b_na_ref,
               W_zr_ref, U_zr_ref, b_zr_ref, Wh_ref, Uh_ref, bh_ref,
               W_gate_ref, b_gate_ref, W_er_ref, b_er_ref,
               W_out_ref, b_out_ref, out_ref):
    g, n = _G, _N
    nodes = nodes_ref[...].reshape(g * n, _H)
    e2 = e2_ref[...].reshape(g * n * n, _E)

    ep = jnp.dot(e2, W_edge_ref[...], preferred_element_type=_F32)
    ep_emb = ep[:, :_MSG].reshape(g, n, n, _MSG).astype(jnp.bfloat16)
    madj = ep[:, 2 * _MSG:].reshape(g, n, n, _MSG)
    ep_att = jnp.where(madj != 0.0,
                       ep[:, _MSG:2 * _MSG].reshape(g, n, n, _MSG),
                       -1e30).astype(jnp.bfloat16)
    node_mask = (jnp.max(madj, axis=2) != 0.0).reshape(g * n, _MSG)

    b_na = b_na_ref[0]
    b_zr = b_zr_ref[0]
    bh = bh_ref[0]
    hidden = nodes
    for _ in range(_PASSES):
        hn = jnp.dot(hidden, W_node_ref[...],
                     preferred_element_type=_F32) + b_na
        hnb = hn.astype(jnp.bfloat16)
        hp = hnb[:, :_MSG].reshape(g, 1, n, _MSG)
        ha = hnb[:, _MSG:].reshape(g, 1, n, _MSG)
        pm = jnp.exp(ha + ep_att)
        emb = jnp.maximum(hp + ep_emb, jnp.bfloat16(0))
        s = jnp.sum(pm, axis=2, dtype=_F32)
        msum = jnp.sum(pm * emb, axis=2, dtype=_F32)
        msgs = (msum / jnp.maximum(s, 1e-30)).reshape(g * n, _MSG)

        zr = jax.nn.sigmoid(
            jnp.dot(msgs, W_zr_ref[...], preferred_element_type=_F32)
            + jnp.dot(hidden, U_zr_ref[...], preferred_element_type=_F32)
            + b_zr)
        z = zr[:, :_H]
        r = zr[:, _H:]
        htil = jnp.tanh(
            jnp.dot(msgs, Wh_ref[...], preferred_element_type=_F32)
            + jnp.dot(r * hidden, Uh_ref[...], preferred_element_type=_F32)
            + bh)
        hidden_new = (1.0 - z) * hidden + z * htil
        hidden = jnp.where(node_mask, hidden_new, hidden)

    gate = jax.nn.sigmoid(
        jnp.dot(hidden, W_gate_ref[:_H, :], preferred_element_type=_F32)
        + jnp.dot(nodes, W_gate_ref[_H:, :], preferred_element_type=_F32)
        + b_gate_ref[0])
    embr = jnp.dot(hidden, W_er_ref[...],
                   preferred_element_type=_F32) + b_er_ref[0]
    gp = jnp.where(node_mask, gate * embr, 0.0).reshape(g, n, _H)
    graph_emb = jnp.sum(gp, axis=1)
    out = jnp.dot(jnp.maximum(graph_emb, 0.0), W_out_ref[...],
                  preferred_element_type=_F32) + b_out_ref[0]
    out_ref[...] = out.reshape(g, 1, _H)


def kernel(nodes, edges, W_emb, b_emb, W_att, b_att, Wz, Uz, bz, Wr, Ur, br,
           Wh, Uh, bh, W_gate, b_gate, W_er, b_er, W_out, b_out):
    e2 = edges.reshape(_B, _N * _N, _E)
    W_edge = jnp.concatenate(
        [W_emb[_H:], W_att[_H:], jnp.ones((_E, _MSG), _F32)], axis=1)
    W_node = jnp.concatenate([W_emb[:_H], W_att[:_H]], axis=1)
    b_na = jnp.concatenate([b_emb, b_att]).reshape(1, 2 * _MSG)
    W_zr = jnp.concatenate([Wz, Wr], axis=1)
    U_zr = jnp.concatenate([Uz, Ur], axis=1)
    b_zr = jnp.concatenate([bz, br]).reshape(1, 2 * _H)
    bh2 = bh.reshape(1, _H)
    b_gate2 = b_gate.reshape(1, _H)
    b_er2 = b_er.reshape(1, _H)
    b_out2 = b_out.reshape(1, _H)

    def full(shape):
        return pl.BlockSpec(shape, lambda b: (0,) * len(shape))

    grid_spec = pl.GridSpec(
        grid=(_B // _G,),
        in_specs=[
            pl.BlockSpec((_G, _N, _H), lambda b: (b, 0, 0)),
            pl.BlockSpec((_G, _N * _N, _E), lambda b: (b, 0, 0)),
            full(W_edge.shape), full(W_node.shape), full(b_na.shape),
            full(W_zr.shape), full(U_zr.shape), full(b_zr.shape),
            full(Wh.shape), full(Uh.shape), full(bh2.shape),
            full(W_gate.shape), full(b_gate2.shape),
            full(W_er.shape), full(b_er2.shape),
            full(W_out.shape), full(b_out2.shape),
        ],
        out_specs=pl.BlockSpec((_G, 1, _H), lambda b: (b, 0, 0)),
    )

    out = pl.pallas_call(
        _mpnn_body,
        grid_spec=grid_spec,
        out_shape=jax.ShapeDtypeStruct((_B, 1, _H), _F32),
        compiler_params=pltpu.CompilerParams(
            dimension_semantics=("arbitrary",)),
    )(nodes, e2, W_edge, W_node, b_na, W_zr, U_zr, b_zr,
      Wh, Uh, bh2, W_gate, b_gate2, W_er, b_er2, W_out, b_out2)
    return out.reshape(_B, _H)


# speedup vs baseline: 6.4266x; 1.1626x over previous
import jax
import jax.numpy as jnp
from jax.experimental import pallas as pl
from jax.experimental.pallas import tpu as pltpu

_B, _N, _H, _E, _MSG, _PASSES = 16, 64, 128, 16, 128, 3
_G = 4
_F32 = jnp.float32


def _mpnn_body(nodes_ref, e2_ref, W_edge_ref, W_node_ref, b_na_ref,
               W_zr_ref, U_zr_ref, b_zr_ref, Wh_ref, Uh_ref, bh_ref,
               W_gate_ref, b_gate_ref, W_er_ref, b_er_ref,
               W_out_ref, b_out_ref, out_ref):
    g, n = _G, _N
    nodes = nodes_ref[...].reshape(g * n, _H)
    e2 = e2_ref[...].reshape(g * n * n, _E)

    ep = jnp.dot(e2, W_edge_ref[...], preferred_element_type=_F32)
    ep_emb = ep[:, :_MSG].reshape(g, n, n, _MSG).astype(jnp.bfloat16)
    madj = ep[:, 2 * _MSG:].reshape(g, n, n, _MSG)
    ep_att = jnp.where(madj != 0.0,
                       ep[:, _MSG:2 * _MSG].reshape(g, n, n, _MSG),
                       -1e30).astype(jnp.bfloat16)
    node_mask = (jnp.max(madj, axis=2) != 0.0).reshape(g * n, _MSG)

    b_na = b_na_ref[0]
    b_zr = b_zr_ref[0]
    bh = bh_ref[0]
    hidden = nodes
    for _ in range(_PASSES):
        hn = jnp.dot(hidden.astype(jnp.bfloat16), W_node_ref[...],
                     preferred_element_type=_F32) + b_na
        hnb = hn.astype(jnp.bfloat16)
        hp = hnb[:, :_MSG].reshape(g, 1, n, _MSG)
        ha = hnb[:, _MSG:].reshape(g, 1, n, _MSG)
        pm = jnp.exp(ha + ep_att)
        emb = jnp.maximum(hp + ep_emb, jnp.bfloat16(0))
        s = jnp.sum(pm, axis=2, dtype=_F32)
        msum = jnp.sum(pm * emb, axis=2, dtype=_F32)
        msgs = (msum / jnp.maximum(s, 1e-30)).reshape(g * n, _MSG)

        zr = jax.nn.sigmoid(
            jnp.dot(msgs, W_zr_ref[...], preferred_element_type=_F32)
            + jnp.dot(hidden, U_zr_ref[...], preferred_element_type=_F32)
            + b_zr)
        z = zr[:, :_H]
        r = zr[:, _H:]
        htil = jnp.tanh(
            jnp.dot(msgs, Wh_ref[...], preferred_element_type=_F32)
            + jnp.dot(r * hidden, Uh_ref[...], preferred_element_type=_F32)
            + bh)
        hidden_new = (1.0 - z) * hidden + z * htil
        hidden = jnp.where(node_mask, hidden_new, hidden)

    gate = jax.nn.sigmoid(
        jnp.dot(hidden, W_gate_ref[:_H, :], preferred_element_type=_F32)
        + jnp.dot(nodes, W_gate_ref[_H:, :], preferred_element_type=_F32)
        + b_gate_ref[0])
    embr = jnp.dot(hidden, W_er_ref[...],
                   preferred_element_type=_F32) + b_er_ref[0]
    gp = jnp.where(node_mask, gate * embr, 0.0).reshape(g, n, _H)
    graph_emb = jnp.sum(gp, axis=1)
    out = jnp.dot(jnp.maximum(graph_emb, 0.0), W_out_ref[...],
                  preferred_element_type=_F32) + b_out_ref[0]
    out_ref[...] = out.reshape(g, 1, _H)


def kernel(nodes, edges, W_emb, b_emb, W_att, b_att, Wz, Uz, bz, Wr, Ur, br,
           Wh, Uh, bh, W_gate, b_gate, W_er, b_er, W_out, b_out):
    e2 = edges.reshape(_B, _N * _N, _E).astype(jnp.bfloat16)
    W_edge = jnp.concatenate(
        [W_emb[_H:], W_att[_H:], jnp.ones((_E, _MSG), _F32)],
        axis=1).astype(jnp.bfloat16)
    W_node = jnp.concatenate([W_emb[:_H], W_att[:_H]],
                             axis=1).astype(jnp.bfloat16)
    b_na = jnp.concatenate([b_emb, b_att]).reshape(1, 2 * _MSG)
    W_zr = jnp.concatenate([Wz, Wr], axis=1)
    U_zr = jnp.concatenate([Uz, Ur], axis=1)
    b_zr = jnp.concatenate([bz, br]).reshape(1, 2 * _H)
    bh2 = bh.reshape(1, _H)
    b_gate2 = b_gate.reshape(1, _H)
    b_er2 = b_er.reshape(1, _H)
    b_out2 = b_out.reshape(1, _H)

    def full(shape):
        return pl.BlockSpec(shape, lambda b: (0,) * len(shape))

    grid_spec = pl.GridSpec(
        grid=(_B // _G,),
        in_specs=[
            pl.BlockSpec((_G, _N, _H), lambda b: (b, 0, 0)),
            pl.BlockSpec((_G, _N * _N, _E), lambda b: (b, 0, 0)),
            full(W_edge.shape), full(W_node.shape), full(b_na.shape),
            full(W_zr.shape), full(U_zr.shape), full(b_zr.shape),
            full(Wh.shape), full(Uh.shape), full(bh2.shape),
            full(W_gate.shape), full(b_gate2.shape),
            full(W_er.shape), full(b_er2.shape),
            full(W_out.shape), full(b_out2.shape),
        ],
        out_specs=pl.BlockSpec((_G, 1, _H), lambda b: (b, 0, 0)),
    )

    out = pl.pallas_call(
        _mpnn_body,
        grid_spec=grid_spec,
        out_shape=jax.ShapeDtypeStruct((_B, 1, _H), _F32),
        compiler_params=pltpu.CompilerParams(
            dimension_semantics=("arbitrary",)),
    )(nodes, e2, W_edge, W_node, b_na, W_zr, U_zr, b_zr,
      Wh, Uh, bh2, W_gate, b_gate2, W_er, b_er2, W_out, b_out2)
    return out.reshape(_B, _H)


# speedup vs baseline: 6.8017x; 1.0584x over previous
import jax
import jax.numpy as jnp
from jax.experimental import pallas as pl
from jax.experimental.pallas import tpu as pltpu

_B, _N, _H, _E, _MSG, _PASSES = 16, 64, 128, 16, 128, 3
_G = 4
_F32 = jnp.float32


def _mpnn_body(nodes_ref, e2_ref, W_edge_ref, W_node_ref, b_na_ref,
               W_zr_ref, U_zr_ref, b_zr_ref, Wh_ref, Uh_ref, bh_ref,
               W_gate_ref, b_gate_ref, W_er_ref, b_er_ref,
               W_out_ref, b_out_ref, out_ref):
    g, n = _G, _N
    nodes = nodes_ref[...].reshape(g * n, _H)
    e2 = e2_ref[...].reshape(g * n * n, _E)

    ep = jnp.dot(e2, W_edge_ref[...], preferred_element_type=_F32)
    ep_emb = ep[:, :_MSG].reshape(g, n, n, _MSG).astype(jnp.bfloat16)
    madj = ep[:, 2 * _MSG:].reshape(g, n, n, _MSG)
    P = jnp.exp(jnp.where(madj != 0.0,
                          ep[:, _MSG:2 * _MSG].reshape(g, n, n, _MSG),
                          -1e30)).astype(jnp.bfloat16)
    node_mask = (jnp.max(madj, axis=2) != 0.0).reshape(g * n, _MSG)

    b_na = b_na_ref[0]
    b_zr = b_zr_ref[0]
    bh = bh_ref[0]
    hidden = nodes
    for _ in range(_PASSES):
        hn = jnp.dot(hidden.astype(jnp.bfloat16), W_node_ref[...],
                     preferred_element_type=_F32) + b_na
        hp = hn[:, :_MSG].astype(jnp.bfloat16).reshape(g, 1, n, _MSG)
        A = jnp.exp(hn[:, _MSG:]).astype(jnp.bfloat16).reshape(g, 1, n, _MSG)
        pm = A * P
        emb = jnp.maximum(hp + ep_emb, jnp.bfloat16(0))
        s = jnp.sum(pm, axis=2, dtype=_F32)
        msum = jnp.sum(pm * emb, axis=2, dtype=_F32)
        msgs = (msum / jnp.maximum(s, 1e-30)).reshape(g * n, _MSG)

        zr = jax.nn.sigmoid(
            jnp.dot(msgs, W_zr_ref[...], preferred_element_type=_F32)
            + jnp.dot(hidden, U_zr_ref[...], preferred_element_type=_F32)
            + b_zr)
        z = zr[:, :_H]
        r = zr[:, _H:]
        htil = jnp.tanh(
            jnp.dot(msgs, Wh_ref[...], preferred_element_type=_F32)
            + jnp.dot(r * hidden, Uh_ref[...], preferred_element_type=_F32)
            + bh)
        hidden_new = (1.0 - z) * hidden + z * htil
        hidden = jnp.where(node_mask, hidden_new, hidden)

    gate = jax.nn.sigmoid(
        jnp.dot(hidden, W_gate_ref[:_H, :], preferred_element_type=_F32)
        + jnp.dot(nodes, W_gate_ref[_H:, :], preferred_element_type=_F32)
        + b_gate_ref[0])
    embr = jnp.dot(hidden, W_er_ref[...],
                   preferred_element_type=_F32) + b_er_ref[0]
    gp = jnp.where(node_mask, gate * embr, 0.0).reshape(g, n, _H)
    graph_emb = jnp.sum(gp, axis=1)
    out = jnp.dot(jnp.maximum(graph_emb, 0.0), W_out_ref[...],
                  preferred_element_type=_F32) + b_out_ref[0]
    out_ref[...] = out.reshape(g, 1, _H)


def kernel(nodes, edges, W_emb, b_emb, W_att, b_att, Wz, Uz, bz, Wr, Ur, br,
           Wh, Uh, bh, W_gate, b_gate, W_er, b_er, W_out, b_out):
    e2 = edges.reshape(_B, _N * _N, _E).astype(jnp.bfloat16)
    W_edge = jnp.concatenate(
        [W_emb[_H:], W_att[_H:], jnp.ones((_E, _MSG), _F32)],
        axis=1).astype(jnp.bfloat16)
    W_node = jnp.concatenate([W_emb[:_H], W_att[:_H]],
                             axis=1).astype(jnp.bfloat16)
    b_na = jnp.concatenate([b_emb, b_att]).reshape(1, 2 * _MSG)
    W_zr = jnp.concatenate([Wz, Wr], axis=1)
    U_zr = jnp.concatenate([Uz, Ur], axis=1)
    b_zr = jnp.concatenate([bz, br]).reshape(1, 2 * _H)
    bh2 = bh.reshape(1, _H)
    b_gate2 = b_gate.reshape(1, _H)
    b_er2 = b_er.reshape(1, _H)
    b_out2 = b_out.reshape(1, _H)

    def full(shape):
        return pl.BlockSpec(shape, lambda b: (0,) * len(shape))

    grid_spec = pl.GridSpec(
        grid=(_B // _G,),
        in_specs=[
            pl.BlockSpec((_G, _N, _H), lambda b: (b, 0, 0)),
            pl.BlockSpec((_G, _N * _N, _E), lambda b: (b, 0, 0)),
            full(W_edge.shape), full(W_node.shape), full(b_na.shape),
            full(W_zr.shape), full(U_zr.shape), full(b_zr.shape),
            full(Wh.shape), full(Uh.shape), full(bh2.shape),
            full(W_gate.shape), full(b_gate2.shape),
            full(W_er.shape), full(b_er2.shape),
            full(W_out.shape), full(b_out2.shape),
        ],
        out_specs=pl.BlockSpec((_G, 1, _H), lambda b: (b, 0, 0)),
    )

    out = pl.pallas_call(
        _mpnn_body,
        grid_spec=grid_spec,
        out_shape=jax.ShapeDtypeStruct((_B, 1, _H), _F32),
        compiler_params=pltpu.CompilerParams(
            dimension_semantics=("arbitrary",)),
    )(nodes, e2, W_edge, W_node, b_na, W_zr, U_zr, b_zr,
      Wh, Uh, bh2, W_gate, b_gate2, W_er, b_er2, W_out, b_out2)
    return out.reshape(_B, _H)
